# baseline (speedup 1.0000x reference)
import functools

import jax
import jax.numpy as jnp
from jax import lax
from jax.experimental import pallas as pl
from jax.experimental.pallas import tpu as pltpu
from jax.experimental.pallas import tpu_sc as plsc

N = 10000
D = 128
E = 320000

NC = 2
NS = 16
NW = NC * NS

WIN = 64
SB = 8
NSB = 20
WPT = SB * NSB
SLOTS = WPT * WIN
E_PAD = SLOTS * NW
NP = 10240
RPT = NP // NS

_mesh = plsc.VectorSubcoreMesh(core_axis_name="c", subcore_axis_name="s")


@functools.partial(
    pl.kernel,
    out_type=jax.ShapeDtypeStruct((NC, NP), jnp.float32),
    mesh=_mesh,
    scratch_types=[
        pltpu.VMEM((WPT, WIN), jnp.int32),
        pltpu.VMEM((WIN,), jnp.float32),
        pltpu.VMEM_SHARED((NP,), jnp.float32),
        pltpu.SemaphoreType.DMA,
    ],
)
def _deg_kernel(dstp_hbm, ones_hbm, zeros1_hbm, degp_hbm, dst_v, ones_v,
                deg_sh, sem):
    c = lax.axis_index("c")
    s = lax.axis_index("s")
    wid = s * NC + c
    pltpu.sync_copy(ones_hbm, ones_v)
    pltpu.sync_copy(zeros1_hbm.at[pl.ds(s * RPT, RPT)],
                    deg_sh.at[pl.ds(s * RPT, RPT)])
    pltpu.sync_copy(dstp_hbm.at[wid], dst_v)
    plsc.subcore_barrier()

    def fire(j, carry):
        pltpu.async_copy(ones_v, deg_sh.at[dst_v.at[j]], sem, add=True)
        return carry

    lax.fori_loop(0, WPT, fire, 0)

    def drain(j, carry):
        pltpu.make_async_copy(ones_v, deg_sh.at[dst_v.at[j]], sem).wait()
        return carry

    lax.fori_loop(0, WPT, drain, 0)
    plsc.subcore_barrier()
    pltpu.sync_copy(deg_sh.at[pl.ds(s * RPT, RPT)],
                    degp_hbm.at[c, pl.ds(s * RPT, RPT)])


@functools.partial(
    pl.kernel,
    out_type=jax.ShapeDtypeStruct((NC, NP, D), jnp.float32),
    mesh=_mesh,
    scratch_types=[
        pltpu.VMEM((SB, WIN), jnp.int32),
        pltpu.VMEM((SB, WIN), jnp.int32),
        pltpu.VMEM((SB, WIN), jnp.int32),
        pltpu.VMEM((SB, WIN), jnp.int32),
        pltpu.VMEM((WIN, D), jnp.float32),
        pltpu.VMEM((WIN, D), jnp.float32),
        pltpu.VMEM((WIN, D), jnp.float32),
        pltpu.VMEM((WIN, D), jnp.float32),
        pltpu.VMEM_SHARED((NP, D), jnp.float32),
        pltpu.SemaphoreType.DMA,
        pltpu.SemaphoreType.DMA,
        pltpu.SemaphoreType.DMA,
        pltpu.SemaphoreType.DMA,
        pltpu.SemaphoreType.DMA,
        pltpu.SemaphoreType.DMA,
        pltpu.SemaphoreType.DMA,
        pltpu.SemaphoreType.DMA,
        pltpu.SemaphoreType.DMA,
        pltpu.SemaphoreType.DMA,
        pltpu.SemaphoreType.DMA,
        pltpu.SemaphoreType.DMA,
    ],
)
def _edge_kernel(h2_hbm, srcp_hbm, dstp_hbm, zeros2_hbm, accp_hbm,
                 ssb_a, ssb_b, dsb_a, dsb_b,
                 rows_0, rows_1, rows_2, rows_3, acc_sh,
                 sem_r0, sem_r1, sem_r2, sem_r3,
                 sem_w0, sem_w1, sem_w2, sem_w3,
                 sem_sa, sem_sb, sem_da, sem_db):
    c = lax.axis_index("c")
    s = lax.axis_index("s")
    wid = s * NC + c
    pltpu.sync_copy(zeros2_hbm.at[pl.ds(s * RPT, RPT)],
                    acc_sh.at[pl.ds(s * RPT, RPT)])
    plsc.subcore_barrier()

    NB = 4
    rows = (rows_0, rows_1, rows_2, rows_3)
    sem_r = (sem_r0, sem_r1, sem_r2, sem_r3)
    sem_w = (sem_w0, sem_w1, sem_w2, sem_w3)

    def start_rows(w, p, idx):
        pltpu.async_copy(h2_hbm.at[idx], rows[p], sem_r[p])

    def wait_rows(p, idx):
        pltpu.make_async_copy(h2_hbm.at[idx], rows[p], sem_r[p]).wait()

    def wait_scat(p, dsb, i):
        pltpu.make_async_copy(rows[p], acc_sh.at[dsb.at[i]],
                              sem_w[p]).wait()

    pltpu.sync_copy(srcp_hbm.at[wid, pl.ds(0, SB)], ssb_a)
    pltpu.async_copy(srcp_hbm.at[wid, pl.ds(SB, SB)], ssb_b, sem_sb)
    pltpu.async_copy(dstp_hbm.at[wid, pl.ds(0, SB)], dsb_a, sem_da)
    for w0 in range(2):
        start_rows(w0, w0, ssb_a.at[w0])

    def body(t, carry):
        for mm_off, ssb, ossb, sem_s, osem_s, dsb, odsb, sem_d, osem_d in (
                (0, ssb_a, ssb_b, sem_sa, sem_sb, dsb_a, dsb_b, sem_da,
                 sem_db),
                (1, ssb_b, ssb_a, sem_sb, sem_sa, dsb_b, dsb_a, sem_db,
                 sem_da)):
            m = 2 * t + mm_off
            pltpu.make_async_copy(
                dstp_hbm.at[wid, pl.ds(m * SB, SB)], dsb, sem_d).wait()
            for j in range(SB):
                w = m * SB + j
                p = j % NB
                wait_rows(p, ssb.at[j])
                pltpu.async_copy(rows[p], acc_sh.at[dsb.at[j]],
                                 sem_w[p], add=True)
                r = (j + 2) % NB

                @pl.when(w + 2 < WPT)
                def _():
                    if j < 2:
                        @pl.when(w >= 2)
                        def _():
                            wait_scat(r, odsb, j + 6)
                    else:
                        wait_scat(r, dsb, j - 2)
                    if j == 6:
                        pltpu.make_async_copy(
                            srcp_hbm.at[wid, pl.ds((m + 1) * SB, SB)],
                            ossb, osem_s).wait()
                    if j < 6:
                        start_rows(w + 2, r, ssb.at[j + 2])
                    else:
                        start_rows(w + 2, r, ossb.at[j - 6])

                if j == 1:
                    @pl.when(m + 1 < NSB)
                    def _():
                        pltpu.async_copy(
                            dstp_hbm.at[wid, pl.ds((m + 1) * SB, SB)],
                            odsb, osem_d)

                if j == 7:
                    @pl.when(m + 2 < NSB)
                    def _():
                        pltpu.async_copy(
                            srcp_hbm.at[wid, pl.ds((m + 2) * SB, SB)],
                            ssb, sem_s)

        return carry

    lax.fori_loop(0, NSB // 2, body, 0)
    for j in range(4, SB):
        wait_scat(j % NB, dsb_b, j)
    plsc.subcore_barrier()
    pltpu.sync_copy(acc_sh.at[pl.ds(s * RPT, RPT)],
                    accp_hbm.at[c, pl.ds(s * RPT, RPT)])


def _h2_body(x_ref, w_ref, degp_ref, h2_ref):
    deg = degp_ref[0, :] + degp_ref[1, :] + 1.0
    dinv = lax.rsqrt(deg)
    h = jnp.dot(x_ref[...], w_ref[...], preferred_element_type=jnp.float32)
    h2_ref[...] = h * dinv[:, None]


def _out_body(accp_ref, h2_ref, degp_ref, b_ref, g_ref, beta_ref, o_ref):
    deg = degp_ref[0, :] + degp_ref[1, :] + 1.0
    dinv = lax.rsqrt(deg)
    pre = (accp_ref[0] + accp_ref[1] + h2_ref[...]) * dinv[:, None] + b_ref[...]
    mean = jnp.mean(pre, axis=1, keepdims=True)
    cent = pre - mean
    var = jnp.mean(cent * cent, axis=1, keepdims=True)
    o_ref[...] = jnp.maximum(
        g_ref[...] * cent * lax.rsqrt(var + 1e-5) + beta_ref[...], 0.0)


_BR = 1024
_GRID = -(-N // _BR)


def kernel(x, edge_index, W, b, ln_gamma, ln_beta):
    src = edge_index[0].astype(jnp.int32)
    dst = edge_index[1].astype(jnp.int32)
    npad = E_PAD - E
    ar = jnp.arange(npad, dtype=jnp.int32)
    srcp = jnp.concatenate([src, ar % N]).reshape(NW, WPT, WIN)
    dstp = jnp.concatenate([dst, N + ar % (NP - N)]).reshape(NW, WPT, WIN)
    zeros1 = jnp.zeros((NP,), jnp.float32)
    zeros2 = jnp.zeros((NP, D), jnp.float32)
    ones = jnp.ones((WIN,), jnp.float32)

    degp = _deg_kernel(dstp, ones, zeros1)

    h2 = pl.pallas_call(
        _h2_body,
        grid=(_GRID,),
        in_specs=[
            pl.BlockSpec((_BR, D), lambda i: (i, 0)),
            pl.BlockSpec((D, D), lambda i: (0, 0)),
            pl.BlockSpec((2, _BR), lambda i: (0, i)),
        ],
        out_specs=pl.BlockSpec((_BR, D), lambda i: (i, 0)),
        out_shape=jax.ShapeDtypeStruct((N, D), jnp.float32),
    )(x, W, degp)

    accp = _edge_kernel(h2, srcp, dstp, zeros2)

    out = pl.pallas_call(
        _out_body,
        grid=(_GRID,),
        in_specs=[
            pl.BlockSpec((2, _BR, D), lambda i: (0, i, 0)),
            pl.BlockSpec((_BR, D), lambda i: (i, 0)),
            pl.BlockSpec((2, _BR), lambda i: (0, i)),
            pl.BlockSpec((1, D), lambda i: (0, 0)),
            pl.BlockSpec((1, D), lambda i: (0, 0)),
            pl.BlockSpec((1, D), lambda i: (0, 0)),
        ],
        out_specs=pl.BlockSpec((_BR, D), lambda i: (i, 0)),
        out_shape=jax.ShapeDtypeStruct((N, D), jnp.float32),
    )(accp, h2, degp, b.reshape(1, D), ln_gamma.reshape(1, D),
      ln_beta.reshape(1, D))
    return out


# speedup vs baseline: 1.1769x; 1.1769x over previous
import functools

import jax
import jax.numpy as jnp
from jax import lax
from jax.experimental import pallas as pl
from jax.experimental.pallas import tpu as pltpu
from jax.experimental.pallas import tpu_sc as plsc

N = 10000
D = 128
E = 320000

NC = 2
NS = 16
NW = NC * NS

WIN = 64
SB = 8
NSB = 20
WPT = SB * NSB
SLOTS = WPT * WIN
E_PAD = SLOTS * NW
NP = 10240
RPT = NP // NS

_mesh = plsc.VectorSubcoreMesh(core_axis_name="c", subcore_axis_name="s")


@functools.partial(
    pl.kernel,
    out_type=jax.ShapeDtypeStruct((NC, NP), jnp.float32),
    mesh=_mesh,
    scratch_types=[
        pltpu.VMEM((WPT, WIN), jnp.int32),
        pltpu.VMEM((WIN,), jnp.float32),
        pltpu.VMEM_SHARED((NP,), jnp.float32),
        pltpu.SemaphoreType.DMA,
    ],
)
def _deg_kernel(dstp_hbm, ones_hbm, zeros1_hbm, degp_hbm, dst_v, ones_v,
                deg_sh, sem):
    c = lax.axis_index("c")
    s = lax.axis_index("s")
    wid = s * NC + c
    pltpu.sync_copy(ones_hbm, ones_v)
    pltpu.sync_copy(zeros1_hbm.at[pl.ds(s * RPT, RPT)],
                    deg_sh.at[pl.ds(s * RPT, RPT)])
    pltpu.sync_copy(dstp_hbm.at[wid], dst_v)
    plsc.subcore_barrier()

    def fire(j, carry):
        pltpu.async_copy(ones_v, deg_sh.at[dst_v.at[j]], sem, add=True)
        return carry

    lax.fori_loop(0, WPT, fire, 0)

    def drain(j, carry):
        pltpu.make_async_copy(ones_v, deg_sh.at[dst_v.at[j]], sem).wait()
        return carry

    lax.fori_loop(0, WPT, drain, 0)
    plsc.subcore_barrier()
    pltpu.sync_copy(deg_sh.at[pl.ds(s * RPT, RPT)],
                    degp_hbm.at[c, pl.ds(s * RPT, RPT)])


@functools.partial(
    pl.kernel,
    out_type=jax.ShapeDtypeStruct((NC, NP, D), jnp.float32),
    mesh=_mesh,
    scratch_types=[
        pltpu.VMEM((SB, WIN), jnp.int32),
        pltpu.VMEM((SB, WIN), jnp.int32),
        pltpu.VMEM((SB, WIN), jnp.int32),
        pltpu.VMEM((SB, WIN), jnp.int32),
        pltpu.VMEM((WIN, D), jnp.float32),
        pltpu.VMEM((WIN, D), jnp.float32),
        pltpu.VMEM((WIN, D), jnp.float32),
        pltpu.VMEM((WIN, D), jnp.float32),
        pltpu.VMEM_SHARED((NP, D), jnp.float32),
        pltpu.SemaphoreType.DMA,
        pltpu.SemaphoreType.DMA,
        pltpu.SemaphoreType.DMA,
        pltpu.SemaphoreType.DMA,
        pltpu.SemaphoreType.DMA,
        pltpu.SemaphoreType.DMA,
        pltpu.SemaphoreType.DMA,
        pltpu.SemaphoreType.DMA,
    ],
)
def _edge_kernel(h2_hbm, srcp_hbm, dstp_hbm, zeros2_hbm, accp_hbm,
                 ssb_a, ssb_b, dsb_a, dsb_b,
                 rows_0, rows_1, rows_2, rows_3, acc_sh,
                 sem_r0, sem_r1, sem_r2, sem_r3,
                 sem_sa, sem_sb, sem_da, sem_db):
    c = lax.axis_index("c")
    s = lax.axis_index("s")
    wid = s * NC + c
    pltpu.sync_copy(zeros2_hbm.at[pl.ds(s * RPT, RPT)],
                    acc_sh.at[pl.ds(s * RPT, RPT)])
    plsc.subcore_barrier()

    NB = 4
    rows = (rows_0, rows_1, rows_2, rows_3)
    sem_r = (sem_r0, sem_r1, sem_r2, sem_r3)

    def start_rows(p, idx):
        pltpu.async_copy(h2_hbm.at[idx], rows[p], sem_r[p])

    def wait_rows(p, idx):
        pltpu.make_async_copy(h2_hbm.at[idx], rows[p], sem_r[p]).wait()

    pltpu.sync_copy(srcp_hbm.at[wid, pl.ds(0, SB)], ssb_a)
    pltpu.async_copy(srcp_hbm.at[wid, pl.ds(SB, SB)], ssb_b, sem_sb)
    pltpu.async_copy(dstp_hbm.at[wid, pl.ds(0, SB)], dsb_a, sem_da)
    pltpu.async_copy(dstp_hbm.at[wid, pl.ds(SB, SB)], dsb_b, sem_db)
    for w0 in range(4):
        start_rows(w0, ssb_a.at[w0])

    def body(t, carry):
        for mm_off, ssb, ossb, sem_s, osem_s, dsb, sem_d in (
                (0, ssb_a, ssb_b, sem_sa, sem_sb, dsb_a, sem_da),
                (1, ssb_b, ssb_a, sem_sb, sem_sa, dsb_b, sem_db)):
            m = 2 * t + mm_off
            pltpu.make_async_copy(
                dstp_hbm.at[wid, pl.ds(m * SB, SB)], dsb, sem_d).wait()
            for j in range(SB):
                w = m * SB + j
                p = j % NB
                wait_rows(p, ssb.at[j])
                pltpu.sync_copy(rows[p], acc_sh.at[dsb.at[j]], add=True)

                @pl.when(w + NB < WPT)
                def _():
                    if j == 4:
                        pltpu.make_async_copy(
                            srcp_hbm.at[wid, pl.ds((m + 1) * SB, SB)],
                            ossb, osem_s).wait()
                    if j < 4:
                        start_rows(p, ssb.at[j + 4])
                    else:
                        start_rows(p, ossb.at[j - 4])

            @pl.when(m + 2 < NSB)
            def _():
                pltpu.async_copy(
                    srcp_hbm.at[wid, pl.ds((m + 2) * SB, SB)], ssb, sem_s)
                pltpu.async_copy(
                    dstp_hbm.at[wid, pl.ds((m + 2) * SB, SB)], dsb, sem_d)

        return carry

    lax.fori_loop(0, NSB // 2, body, 0)
    plsc.subcore_barrier()
    pltpu.sync_copy(acc_sh.at[pl.ds(s * RPT, RPT)],
                    accp_hbm.at[c, pl.ds(s * RPT, RPT)])


def _h2_body(x_ref, w_ref, degp_ref, h2_ref):
    deg = degp_ref[0, :] + degp_ref[1, :] + 1.0
    dinv = lax.rsqrt(deg)
    h = jnp.dot(x_ref[...], w_ref[...], preferred_element_type=jnp.float32)
    h2_ref[...] = h * dinv[:, None]


def _out_body(accp_ref, h2_ref, degp_ref, b_ref, g_ref, beta_ref, o_ref):
    deg = degp_ref[0, :] + degp_ref[1, :] + 1.0
    dinv = lax.rsqrt(deg)
    pre = (accp_ref[0] + accp_ref[1] + h2_ref[...]) * dinv[:, None] + b_ref[...]
    mean = jnp.mean(pre, axis=1, keepdims=True)
    cent = pre - mean
    var = jnp.mean(cent * cent, axis=1, keepdims=True)
    o_ref[...] = jnp.maximum(
        g_ref[...] * cent * lax.rsqrt(var + 1e-5) + beta_ref[...], 0.0)


_BR = 1024
_GRID = -(-N // _BR)


def kernel(x, edge_index, W, b, ln_gamma, ln_beta):
    src = edge_index[0].astype(jnp.int32)
    dst = edge_index[1].astype(jnp.int32)
    npad = E_PAD - E
    ar = jnp.arange(npad, dtype=jnp.int32)
    srcp = jnp.concatenate([src, ar % N]).reshape(NW, WPT, WIN)
    dstp = jnp.concatenate([dst, N + ar % (NP - N)]).reshape(NW, WPT, WIN)
    zeros1 = jnp.zeros((NP,), jnp.float32)
    zeros2 = jnp.zeros((NP, D), jnp.float32)
    ones = jnp.ones((WIN,), jnp.float32)

    degp = _deg_kernel(dstp, ones, zeros1)

    h2 = pl.pallas_call(
        _h2_body,
        grid=(_GRID,),
        in_specs=[
            pl.BlockSpec((_BR, D), lambda i: (i, 0)),
            pl.BlockSpec((D, D), lambda i: (0, 0)),
            pl.BlockSpec((2, _BR), lambda i: (0, i)),
        ],
        out_specs=pl.BlockSpec((_BR, D), lambda i: (i, 0)),
        out_shape=jax.ShapeDtypeStruct((N, D), jnp.float32),
    )(x, W, degp)

    accp = _edge_kernel(h2, srcp, dstp, zeros2)

    out = pl.pallas_call(
        _out_body,
        grid=(_GRID,),
        in_specs=[
            pl.BlockSpec((2, _BR, D), lambda i: (0, i, 0)),
            pl.BlockSpec((_BR, D), lambda i: (i, 0)),
            pl.BlockSpec((2, _BR), lambda i: (0, i)),
            pl.BlockSpec((1, D), lambda i: (0, 0)),
            pl.BlockSpec((1, D), lambda i: (0, 0)),
            pl.BlockSpec((1, D), lambda i: (0, 0)),
        ],
        out_specs=pl.BlockSpec((_BR, D), lambda i: (i, 0)),
        out_shape=jax.ShapeDtypeStruct((N, D), jnp.float32),
    )(accp, h2, degp, b.reshape(1, D), ln_gamma.reshape(1, D),
      ln_beta.reshape(1, D))
    return out
